# baseline (speedup 1.0000x reference)
import functools

import jax
import jax.numpy as jnp
from jax import lax
from jax.experimental import pallas as pl
from jax.experimental.pallas import tpu as pltpu
from jax.experimental.pallas import tpu_sc as plsc

N_ROWS = 10000
D = 128
W = D // 2
N_EDGES = 320000
NC = 2
NS = 16
NW = NC * NS
E_PER_W = N_EDGES // NW
C = 80
N_CHUNKS = E_PER_W // C
NB = 4


def _normalize_table(x):
    def body(x_ref, o_ref):
        v = x_ref[...]
        s = jnp.sum(v * v, axis=-1, keepdims=True)
        n = v * lax.rsqrt(jnp.maximum(s, 1e-24))
        u = lax.bitcast_convert_type(n, jnp.int32)
        rne = lax.shift_right_logical(
            u + 0x7FFF + (lax.shift_right_logical(u, 16) & 1), 16)
        o_ref[...] = rne[:, :W] | lax.shift_left(rne[:, W:], 16)

    blk = N_ROWS // 5
    return pl.pallas_call(
        body,
        grid=(5,),
        in_specs=[pl.BlockSpec((blk, D), lambda i: (i, 0))],
        out_specs=pl.BlockSpec((blk, W), lambda i: (i, 0)),
        out_shape=jax.ShapeDtypeStruct((x.shape[0], W), jnp.int32),
    )(x)


_mesh = plsc.VectorSubcoreMesh(core_axis_name="c", subcore_axis_name="s")

_GATHER_DNUMS = lax.GatherDimensionNumbers(
    offset_dims=(), collapsed_slice_dims=(0,), start_index_map=(0,))


def _lane_shuffle(v, perm):
    return lax.gather(
        v, perm.reshape(16, 1), _GATHER_DNUMS, (1,),
        mode=lax.GatherScatterMode.PROMISE_IN_BOUNDS)


@functools.partial(
    pl.kernel,
    mesh=_mesh,
    out_type=jax.ShapeDtypeStruct((N_EDGES,), jnp.float32),
    compiler_params=pltpu.CompilerParams(use_tc_tiling_on_sc=False),
    scratch_types=[
        pltpu.VMEM((E_PER_W,), jnp.int32),
        pltpu.VMEM((E_PER_W,), jnp.int32),
        [pltpu.VMEM((C, W), jnp.int32) for _ in range(NB)],
        [pltpu.VMEM((C, W), jnp.int32) for _ in range(NB)],
        pltpu.VMEM((E_PER_W,), jnp.float32),
        pltpu.VMEM((C, 16), jnp.float32),
        pltpu.VMEM_SHARED((N_ROWS, W), jnp.int32),
        [pltpu.SemaphoreType.DMA for _ in range(NB)],
        [pltpu.SemaphoreType.DMA for _ in range(NB)],
    ],
)
def _sc_gather_dot(xn_hbm, eli_hbm, out_hbm,
                   idx_s_all, idx_d_all,
                   rows_s, rows_d, out_v, out_wide, table_sp,
                   sems_s, sems_d):
    wid = lax.axis_index("s") * NC + lax.axis_index("c")
    base = wid * E_PER_W

    cps = pltpu.async_copy(
        eli_hbm.at[0, pl.ds(base, E_PER_W)], idx_s_all, sems_s[0])
    cpd = pltpu.async_copy(
        eli_hbm.at[1, pl.ds(base, E_PER_W)], idx_d_all, sems_d[0])

    @pl.when(lax.axis_index("s") == 0)
    def _():
        pltpu.sync_copy(xn_hbm, table_sp)

    cps.wait()
    cpd.wait()
    plsc.subcore_barrier()

    lane = lax.iota(jnp.int32, 16)
    himask = jnp.int32(-65536)

    def issue(c, t):
        pltpu.async_copy(
            table_sp.at[idx_s_all.at[pl.ds(c * C, C)]], rows_s[t], sems_s[t])
        pltpu.async_copy(
            table_sp.at[idx_d_all.at[pl.ds(c * C, C)]], rows_d[t], sems_d[t])

    def wait(c, t):
        pltpu.make_async_copy(
            table_sp.at[idx_s_all.at[pl.ds(c * C, C)]], rows_s[t], sems_s[t]).wait()
        pltpu.make_async_copy(
            table_sp.at[idx_d_all.at[pl.ds(c * C, C)]], rows_d[t], sems_d[t]).wait()

    def compute(c, t):
        obase = c * C
        rs = rows_s[t]
        rd = rows_d[t]

        @plsc.parallel_loop(0, C, unroll=2)
        def _edge(e):
            q = jnp.zeros((16,), jnp.float32)
            for k in range(W // 16):
                ws = rs[e, pl.ds(16 * k, 16)]
                wd = rd[e, pl.ds(16 * k, 16)]
                a_s = lax.bitcast_convert_type(lax.shift_left(ws, 16), jnp.float32)
                a_d = lax.bitcast_convert_type(lax.shift_left(wd, 16), jnp.float32)
                b_s = lax.bitcast_convert_type(ws & himask, jnp.float32)
                b_d = lax.bitcast_convert_type(wd & himask, jnp.float32)
                q = q + a_s * a_d
                q = q + b_s * b_d
            for m in (8, 4, 2, 1):
                q = q + _lane_shuffle(q, lane ^ m)
            out_wide[e, :] = q

        @plsc.parallel_loop(0, C // 16)
        def _compact(g):
            vec = out_wide[g * 16, :]
            for l in range(1, 16):
                vec = jnp.where(lane == l, out_wide[g * 16 + l, :], vec)
            out_v[pl.ds(obase + g * 16, 16)] = vec

    for i in range(NB):
        issue(i, i)

    def multi_body(j, carry):
        for t in range(NB):
            c = NB * j + t
            wait(c, t)
            compute(c, t)

            @pl.when(c + NB < N_CHUNKS)
            def _(c=c, t=t):
                issue(c + NB, t)
        return carry

    lax.fori_loop(0, N_CHUNKS // NB, multi_body, 0)

    last = N_CHUNKS - 1
    wait(last, last % NB)
    compute(last, last % NB)

    pltpu.sync_copy(out_v, out_hbm.at[pl.ds(base, E_PER_W)])


def kernel(x, edge_label_index):
    xn = _normalize_table(x)
    return _sc_gather_dot(xn, edge_label_index)


# speedup vs baseline: 1.0103x; 1.0103x over previous
import functools

import jax
import jax.numpy as jnp
from jax import lax
from jax.experimental import pallas as pl
from jax.experimental.pallas import tpu as pltpu
from jax.experimental.pallas import tpu_sc as plsc

N_ROWS = 10000
D = 128
W = D // 2
N_EDGES = 320000
NC = 2
NS = 16
NW = NC * NS
E_PER_W = N_EDGES // NW
C = 80
N_CHUNKS = E_PER_W // C
NB = 4


def _normalize_table(x):
    def body(x_ref, o_ref):
        v = x_ref[...]
        s = jnp.sum(v * v, axis=-1, keepdims=True)
        n = v * lax.rsqrt(jnp.maximum(s, 1e-24))
        u = lax.bitcast_convert_type(n, jnp.int32)
        rne = lax.shift_right_logical(
            u + 0x7FFF + (lax.shift_right_logical(u, 16) & 1), 16)
        o_ref[...] = rne[:, :W] | lax.shift_left(rne[:, W:], 16)

    blk = N_ROWS // 5
    return pl.pallas_call(
        body,
        grid=(5,),
        in_specs=[pl.BlockSpec((blk, D), lambda i: (i, 0))],
        out_specs=pl.BlockSpec((blk, W), lambda i: (i, 0)),
        out_shape=jax.ShapeDtypeStruct((x.shape[0], W), jnp.int32),
    )(x)


_mesh = plsc.VectorSubcoreMesh(core_axis_name="c", subcore_axis_name="s")

_GATHER_DNUMS = lax.GatherDimensionNumbers(
    offset_dims=(), collapsed_slice_dims=(0,), start_index_map=(0,))


def _lane_shuffle(v, perm):
    return lax.gather(
        v, perm.reshape(16, 1), _GATHER_DNUMS, (1,),
        mode=lax.GatherScatterMode.PROMISE_IN_BOUNDS)


@functools.partial(
    pl.kernel,
    mesh=_mesh,
    out_type=jax.ShapeDtypeStruct((N_EDGES,), jnp.float32),
    compiler_params=pltpu.CompilerParams(use_tc_tiling_on_sc=False),
    scratch_types=[
        pltpu.VMEM((E_PER_W,), jnp.int32),
        pltpu.VMEM((E_PER_W,), jnp.int32),
        [pltpu.VMEM((C, W), jnp.int32) for _ in range(NB)],
        [pltpu.VMEM((C, W), jnp.int32) for _ in range(NB)],
        pltpu.VMEM((E_PER_W,), jnp.float32),
        pltpu.VMEM((C, 16), jnp.float32),
        [pltpu.SemaphoreType.DMA for _ in range(NB)],
        [pltpu.SemaphoreType.DMA for _ in range(NB)],
    ],
)
def _sc_gather_dot(xn_hbm, eli_hbm, out_hbm,
                   idx_s_all, idx_d_all,
                   rows_s, rows_d, out_v, out_wide,
                   sems_s, sems_d):
    wid = lax.axis_index("s") * NC + lax.axis_index("c")
    base = wid * E_PER_W

    cps = pltpu.async_copy(
        eli_hbm.at[0, pl.ds(base, E_PER_W)], idx_s_all, sems_s[0])
    cpd = pltpu.async_copy(
        eli_hbm.at[1, pl.ds(base, E_PER_W)], idx_d_all, sems_d[0])
    cps.wait()
    cpd.wait()

    lane = lax.iota(jnp.int32, 16)
    himask = jnp.int32(-65536)

    def issue(c, t):
        pltpu.async_copy(
            xn_hbm.at[idx_s_all.at[pl.ds(c * C, C)]], rows_s[t], sems_s[t])
        pltpu.async_copy(
            xn_hbm.at[idx_d_all.at[pl.ds(c * C, C)]], rows_d[t], sems_d[t])

    def wait(c, t):
        pltpu.make_async_copy(
            xn_hbm.at[idx_s_all.at[pl.ds(c * C, C)]], rows_s[t], sems_s[t]).wait()
        pltpu.make_async_copy(
            xn_hbm.at[idx_d_all.at[pl.ds(c * C, C)]], rows_d[t], sems_d[t]).wait()

    def compute(c, t):
        obase = c * C
        rs = rows_s[t]
        rd = rows_d[t]

        @plsc.parallel_loop(0, C, unroll=2)
        def _edge(e):
            q = jnp.zeros((16,), jnp.float32)
            for k in range(W // 16):
                ws = rs[e, pl.ds(16 * k, 16)]
                wd = rd[e, pl.ds(16 * k, 16)]
                a_s = lax.bitcast_convert_type(lax.shift_left(ws, 16), jnp.float32)
                a_d = lax.bitcast_convert_type(lax.shift_left(wd, 16), jnp.float32)
                b_s = lax.bitcast_convert_type(ws & himask, jnp.float32)
                b_d = lax.bitcast_convert_type(wd & himask, jnp.float32)
                q = q + a_s * a_d
                q = q + b_s * b_d
            for m in (8, 4, 2, 1):
                q = q + _lane_shuffle(q, lane ^ m)
            out_wide[e, :] = q

        @plsc.parallel_loop(0, C // 16)
        def _compact(g):
            vec = out_wide[g * 16, :]
            for l in range(1, 16):
                vec = jnp.where(lane == l, out_wide[g * 16 + l, :], vec)
            out_v[pl.ds(obase + g * 16, 16)] = vec

    for i in range(NB):
        issue(i, i)

    def multi_body(j, carry):
        for t in range(NB):
            c = NB * j + t
            wait(c, t)
            compute(c, t)

            @pl.when(c + NB < N_CHUNKS)
            def _(c=c, t=t):
                issue(c + NB, t)
        return carry

    lax.fori_loop(0, N_CHUNKS // NB, multi_body, 0)

    last = N_CHUNKS - 1
    wait(last, last % NB)
    compute(last, last % NB)

    pltpu.sync_copy(out_v, out_hbm.at[pl.ds(base, E_PER_W)])


def kernel(x, edge_label_index):
    xn = _normalize_table(x)
    return _sc_gather_dot(xn, edge_label_index)
